# baseline (speedup 1.0000x reference)
import functools

import jax
import jax.numpy as jnp
from jax import lax
from jax.experimental import pallas as pl
from jax.experimental.pallas import tpu as pltpu, tpu_sc as plsc

N = 10000
E = 320000
NUM_GRAPHS = 64
NUM_CLASSES = 10

NC, NS, LANES = 2, 16, 16
NW = NC * NS
EPT = E // NW
K = 64
CPT = 160
EP = NW * CPT * K
NPAD = N + 8
_NB = {16: 8, 32: 8, 64: 8, 128: 4}
RPT = 1000
NWB = N // RPT

_MESH = plsc.VectorSubcoreMesh(core_axis_name="c", subcore_axis_name="s")


def _make_sc_scatter(D):
    NB = _NB[D]

    @functools.partial(
        pl.kernel,
        out_type=jax.ShapeDtypeStruct((NC * N, D), jnp.float32),
        mesh=_MESH,
        scratch_types=[
            [pltpu.VMEM((2, K), jnp.int32) for _ in range(NB)],
            [pltpu.VMEM((K, D), jnp.float32) for _ in range(NB)],
            pltpu.VMEM_SHARED((NPAD, D), jnp.float32),
            [pltpu.SemaphoreType.DMA for _ in range(NB)],
            [pltpu.SemaphoreType.DMA for _ in range(NB)],
        ],
        compiler_params=pltpu.CompilerParams(use_tc_tiling_on_sc=False),
    )
    def sc_scatter(g_hbm, packed_hbm, zeros_hbm, out_hbm,
                   ibuf, rows, acc, isem, gsem):
        c = lax.axis_index("c")
        s = lax.axis_index("s")
        wid = c * NS + s
        r0 = s * RPT
        cbase = wid * CPT

        @pl.when(s < NWB)
        def _zero():
            pltpu.sync_copy(zeros_hbm, acc.at[pl.ds(r0, RPT)])

        def load_idx(chunk, b):
            pltpu.async_copy(packed_hbm.at[cbase + chunk], ibuf[b], isem[b])

        def wait_idx(b):
            pltpu.make_async_copy(packed_hbm.at[cbase], ibuf[b],
                                  isem[b]).wait()

        def gather(b):
            pltpu.async_copy(g_hbm.at[ibuf[b].at[0]], rows[b], gsem[b])

        def wait_gather(b):
            pltpu.make_async_copy(g_hbm.at[ibuf[b].at[0]], rows[b],
                                  gsem[b]).wait()

        for b in range(NB):
            load_idx(b, b)
        for b in range(NB - 1):
            wait_idx(b)
            gather(b)
        plsc.subcore_barrier()

        def outer(j, carry):
            for b in range(NB):
                i = NB * j + b
                bprev = (b - 1) % NB
                wait_gather(b)

                @pl.when(i + NB - 1 < CPT)
                def _issue_gather():
                    wait_idx(bprev)
                    gather(bprev)

                pltpu.sync_copy(rows[b], acc.at[ibuf[b].at[1]], add=True)

                @pl.when(i + NB < CPT)
                def _prefetch_idx():
                    load_idx(i + NB, b)
            return carry

        lax.fori_loop(0, CPT // NB, outer, 0)
        plsc.subcore_barrier()

        @pl.when(s < NWB)
        def _writeback():
            pltpu.sync_copy(acc.at[pl.ds(r0, RPT)],
                            out_hbm.at[pl.ds(c * N + r0, RPT)])

    return sc_scatter


RB = 1000


def _tc_first(dp, x, W1):
    D = W1.shape[1]

    def body(dp_ref, x_ref, w_ref, g_ref, dinv_ref):
        d = dp_ref[...]
        deg = d[0, :, 0] + d[1, :, 0] + 1.0
        dinv = lax.rsqrt(deg)
        h = jnp.dot(x_ref[...], w_ref[...], preferred_element_type=jnp.float32)
        g_ref[...] = h * dinv[:, None]
        dinv_ref[...] = dinv[:, None]

    return pl.pallas_call(
        body,
        grid=(N // RB,),
        in_specs=[
            pl.BlockSpec((2, RB, LANES), lambda i: (0, i, 0)),
            pl.BlockSpec((RB, x.shape[1]), lambda i: (i, 0)),
            pl.BlockSpec(W1.shape, lambda i: (0, 0)),
        ],
        out_specs=[
            pl.BlockSpec((RB, D), lambda i: (i, 0)),
            pl.BlockSpec((RB, 1), lambda i: (i, 0)),
        ],
        out_shape=[
            jax.ShapeDtypeStruct((N, D), jnp.float32),
            jax.ShapeDtypeStruct((N, 1), jnp.float32),
        ],
    )(dp, x, W1)


def _tc_mid(s, g, dinv, b, W):
    D = g.shape[1]
    Do = W.shape[1]

    def body(s_ref, g_ref, dinv_ref, b_ref, w_ref, o_ref):
        sp = s_ref[...]
        dv = dinv_ref[...]
        xn = jnp.maximum(dv * (sp[0] + sp[1] + g_ref[...]) + b_ref[...], 0.0)
        h = jnp.dot(xn, w_ref[...], preferred_element_type=jnp.float32)
        o_ref[...] = h * dv

    return pl.pallas_call(
        body,
        grid=(N // RB,),
        in_specs=[
            pl.BlockSpec((2, RB, D), lambda i: (0, i, 0)),
            pl.BlockSpec((RB, D), lambda i: (i, 0)),
            pl.BlockSpec((RB, 1), lambda i: (i, 0)),
            pl.BlockSpec((1, D), lambda i: (0, 0)),
            pl.BlockSpec(W.shape, lambda i: (0, 0)),
        ],
        out_specs=pl.BlockSpec((RB, Do), lambda i: (i, 0)),
        out_shape=jax.ShapeDtypeStruct((N, Do), jnp.float32),
    )(s, g, dinv, b, W)


def _tc_final(s, g, dinv, b, batch, Wfc, bfc):
    D = g.shape[1]

    def body(s_ref, g_ref, dinv_ref, b_ref, bt_ref, wfc_ref, bfc_ref,
             o_ref, pooled_ref):
        sp = s_ref[...]
        x4 = jnp.maximum(
            dinv_ref[...] * (sp[0] + sp[1] + g_ref[...]) + b_ref[...], 0.0)
        bt = bt_ref[...]

        def seg(gi, carry):
            m = bt == gi
            v = jnp.max(jnp.where(m, x4, -jnp.inf), axis=0, keepdims=True)
            pooled_ref[pl.ds(gi, 1), :] = v
            return carry

        lax.fori_loop(0, NUM_GRAPHS, seg, 0)
        logits = jnp.dot(pooled_ref[...], wfc_ref[...],
                         preferred_element_type=jnp.float32) + bfc_ref[...]
        mx = jnp.max(logits, axis=1, keepdims=True)
        sh = logits - mx
        o_ref[...] = sh - jnp.log(jnp.sum(jnp.exp(sh), axis=1, keepdims=True))

    return pl.pallas_call(
        body,
        grid=(1,),
        in_specs=[
            pl.BlockSpec((2, N, D), lambda i: (0, 0, 0)),
            pl.BlockSpec((N, D), lambda i: (0, 0)),
            pl.BlockSpec((N, 1), lambda i: (0, 0)),
            pl.BlockSpec((1, D), lambda i: (0, 0)),
            pl.BlockSpec((N, 1), lambda i: (0, 0)),
            pl.BlockSpec(Wfc.shape, lambda i: (0, 0)),
            pl.BlockSpec((1, NUM_CLASSES), lambda i: (0, 0)),
        ],
        out_specs=pl.BlockSpec((NUM_GRAPHS, NUM_CLASSES), lambda i: (0, 0)),
        out_shape=jax.ShapeDtypeStruct((NUM_GRAPHS, NUM_CLASSES), jnp.float32),
        scratch_shapes=[pltpu.VMEM((NUM_GRAPHS, D), jnp.float32)],
    )(s, g, dinv, b, batch, Wfc, bfc)


def kernel(x, edge_index, batch, W1, b1, W2, b2, W3, b3, Wfc, bfc):
    npad = EP - E
    pad = jnp.concatenate(
        [jnp.zeros((1, npad), jnp.int32),
         jnp.full((1, npad), N, jnp.int32)], axis=0)
    packed = (jnp.concatenate([edge_index, pad], axis=1)
              .reshape(2, NW * CPT, K).transpose(1, 0, 2))

    dp = _make_sc_scatter(LANES)(
        jnp.ones((N, LANES), jnp.float32), packed,
        jnp.zeros((RPT, LANES), jnp.float32)).reshape(2, N, LANES)
    g1, dinv = _tc_first(dp, x, W1)

    s1 = _make_sc_scatter(128)(g1, packed, jnp.zeros((RPT, 128), jnp.float32))
    g2 = _tc_mid(s1.reshape(2, N, 128), g1, dinv, b1.reshape(1, -1), W2)

    s2 = _make_sc_scatter(64)(g2, packed, jnp.zeros((RPT, 64), jnp.float32))
    g3 = _tc_mid(s2.reshape(2, N, 64), g2, dinv, b2.reshape(1, -1), W3)

    s3 = _make_sc_scatter(32)(g3, packed, jnp.zeros((RPT, 32), jnp.float32))
    return _tc_final(s3.reshape(2, N, 32), g3, dinv, b3.reshape(1, -1),
                     batch.reshape(N, 1), Wfc, bfc.reshape(1, NUM_CLASSES))


# speedup vs baseline: 1.1735x; 1.1735x over previous
import functools

import jax
import jax.numpy as jnp
from jax import lax
from jax.experimental import pallas as pl
from jax.experimental.pallas import tpu as pltpu, tpu_sc as plsc

N = 10000
E = 320000
NUM_GRAPHS = 64
NUM_CLASSES = 10

NC, NS, LANES = 2, 16, 16
NW = NC * NS
EPT = E // NW
K = 128
NCHUNK = 2560
EP = NCHUNK * K
NPAD = N + 8
_NB = {16: 5, 32: 5, 64: 4, 128: 2}
_CPT0 = {16: 80, 32: 110, 64: 116, 128: 122}
RPT = 1000
NWB = N // RPT

_MESH = plsc.VectorSubcoreMesh(core_axis_name="c", subcore_axis_name="s")


def _make_sc_scatter(D):
    NB = _NB[D]
    CPT0 = _CPT0[D]
    CPT1 = 160 - CPT0
    assert CPT0 % NB == 0 and CPT1 % NB == 0

    @functools.partial(
        pl.kernel,
        out_type=jax.ShapeDtypeStruct((NC * N, D), jnp.float32),
        mesh=_MESH,
        scratch_types=[
            [pltpu.VMEM((2, K), jnp.int32) for _ in range(NB)],
            [pltpu.VMEM((K, D), jnp.float32) for _ in range(NB)],
            pltpu.VMEM_SHARED((NPAD, D), jnp.float32),
            [pltpu.SemaphoreType.DMA for _ in range(NB)],
            [pltpu.SemaphoreType.DMA for _ in range(NB)],
        ],
        compiler_params=pltpu.CompilerParams(use_tc_tiling_on_sc=False),
    )
    def sc_scatter(g_hbm, packed_hbm, zeros_hbm, out_hbm,
                   ibuf, rows, acc, isem, gsem):
        c = lax.axis_index("c")
        s = lax.axis_index("s")
        r0 = s * RPT
        cpt = CPT0 - (CPT0 - CPT1) * c
        cbase = c * NS * CPT0 + s * cpt

        @pl.when(s < NWB)
        def _zero():
            pltpu.sync_copy(zeros_hbm, acc.at[pl.ds(r0, RPT)])

        def load_idx(chunk, b):
            pltpu.async_copy(packed_hbm.at[cbase + chunk], ibuf[b], isem[b])

        def wait_idx(b):
            pltpu.make_async_copy(packed_hbm.at[cbase], ibuf[b],
                                  isem[b]).wait()

        def gather(b):
            pltpu.async_copy(g_hbm.at[ibuf[b].at[0]], rows[b], gsem[b])

        def wait_gather(b):
            pltpu.make_async_copy(g_hbm.at[ibuf[b].at[0]], rows[b],
                                  gsem[b]).wait()

        for b in range(NB):
            load_idx(b, b)
        for b in range(NB - 1):
            wait_idx(b)
            gather(b)
        plsc.subcore_barrier()

        def outer(j, carry):
            for b in range(NB):
                i = NB * j + b
                bprev = (b - 1) % NB
                wait_gather(b)

                @pl.when(i + NB - 1 < cpt)
                def _issue_gather():
                    wait_idx(bprev)
                    gather(bprev)

                pltpu.sync_copy(rows[b], acc.at[ibuf[b].at[1]], add=True)

                @pl.when(i + NB < cpt)
                def _prefetch_idx():
                    load_idx(i + NB, b)
            return carry

        lax.fori_loop(0, cpt // NB, outer, 0)
        plsc.subcore_barrier()

        @pl.when(s < NWB)
        def _writeback():
            pltpu.sync_copy(acc.at[pl.ds(r0, RPT)],
                            out_hbm.at[pl.ds(c * N + r0, RPT)])

    return sc_scatter


RB = 1000


def _tc_first(dp, x, W1):
    D = W1.shape[1]

    def body(dp_ref, x_ref, w_ref, g_ref, dinv_ref):
        d = dp_ref[...]
        deg = d[0, :, 0] + d[1, :, 0] + 1.0
        dinv = lax.rsqrt(deg)
        h = jnp.dot(x_ref[...], w_ref[...], preferred_element_type=jnp.float32)
        g_ref[...] = h * dinv[:, None]
        dinv_ref[...] = dinv[:, None]

    return pl.pallas_call(
        body,
        grid=(N // RB,),
        in_specs=[
            pl.BlockSpec((2, RB, LANES), lambda i: (0, i, 0)),
            pl.BlockSpec((RB, x.shape[1]), lambda i: (i, 0)),
            pl.BlockSpec(W1.shape, lambda i: (0, 0)),
        ],
        out_specs=[
            pl.BlockSpec((RB, D), lambda i: (i, 0)),
            pl.BlockSpec((RB, 1), lambda i: (i, 0)),
        ],
        out_shape=[
            jax.ShapeDtypeStruct((N, D), jnp.float32),
            jax.ShapeDtypeStruct((N, 1), jnp.float32),
        ],
    )(dp, x, W1)


def _tc_mid(s, g, dinv, b, W):
    D = g.shape[1]
    Do = W.shape[1]

    def body(s_ref, g_ref, dinv_ref, b_ref, w_ref, o_ref):
        sp = s_ref[...]
        dv = dinv_ref[...]
        xn = jnp.maximum(dv * (sp[0] + sp[1] + g_ref[...]) + b_ref[...], 0.0)
        h = jnp.dot(xn, w_ref[...], preferred_element_type=jnp.float32)
        o_ref[...] = h * dv

    return pl.pallas_call(
        body,
        grid=(N // RB,),
        in_specs=[
            pl.BlockSpec((2, RB, D), lambda i: (0, i, 0)),
            pl.BlockSpec((RB, D), lambda i: (i, 0)),
            pl.BlockSpec((RB, 1), lambda i: (i, 0)),
            pl.BlockSpec((1, D), lambda i: (0, 0)),
            pl.BlockSpec(W.shape, lambda i: (0, 0)),
        ],
        out_specs=pl.BlockSpec((RB, Do), lambda i: (i, 0)),
        out_shape=jax.ShapeDtypeStruct((N, Do), jnp.float32),
    )(s, g, dinv, b, W)


def _tc_final(s, g, dinv, b, batch, Wfc, bfc):
    D = g.shape[1]

    def body(s_ref, g_ref, dinv_ref, b_ref, bt_ref, wfc_ref, bfc_ref,
             o_ref, pooled_ref):
        sp = s_ref[...]
        x4 = jnp.maximum(
            dinv_ref[...] * (sp[0] + sp[1] + g_ref[...]) + b_ref[...], 0.0)
        bt = bt_ref[...]

        def seg(gi, carry):
            m = bt == gi
            v = jnp.max(jnp.where(m, x4, -jnp.inf), axis=0, keepdims=True)
            pooled_ref[pl.ds(gi, 1), :] = v
            return carry

        lax.fori_loop(0, NUM_GRAPHS, seg, 0)
        logits = jnp.dot(pooled_ref[...], wfc_ref[...],
                         preferred_element_type=jnp.float32) + bfc_ref[...]
        mx = jnp.max(logits, axis=1, keepdims=True)
        sh = logits - mx
        o_ref[...] = sh - jnp.log(jnp.sum(jnp.exp(sh), axis=1, keepdims=True))

    return pl.pallas_call(
        body,
        grid=(1,),
        in_specs=[
            pl.BlockSpec((2, N, D), lambda i: (0, 0, 0)),
            pl.BlockSpec((N, D), lambda i: (0, 0)),
            pl.BlockSpec((N, 1), lambda i: (0, 0)),
            pl.BlockSpec((1, D), lambda i: (0, 0)),
            pl.BlockSpec((N, 1), lambda i: (0, 0)),
            pl.BlockSpec(Wfc.shape, lambda i: (0, 0)),
            pl.BlockSpec((1, NUM_CLASSES), lambda i: (0, 0)),
        ],
        out_specs=pl.BlockSpec((NUM_GRAPHS, NUM_CLASSES), lambda i: (0, 0)),
        out_shape=jax.ShapeDtypeStruct((NUM_GRAPHS, NUM_CLASSES), jnp.float32),
        scratch_shapes=[pltpu.VMEM((NUM_GRAPHS, D), jnp.float32)],
    )(s, g, dinv, b, batch, Wfc, bfc)


def kernel(x, edge_index, batch, W1, b1, W2, b2, W3, b3, Wfc, bfc):
    npad = EP - E
    pad = jnp.concatenate(
        [jnp.zeros((1, npad), jnp.int32),
         jnp.full((1, npad), N, jnp.int32)], axis=0)
    packed = (jnp.concatenate([edge_index, pad], axis=1)
              .reshape(2, NCHUNK, K).transpose(1, 0, 2))

    dp = _make_sc_scatter(LANES)(
        jnp.ones((N, LANES), jnp.float32), packed,
        jnp.zeros((RPT, LANES), jnp.float32)).reshape(2, N, LANES)
    g1, dinv = _tc_first(dp, x, W1)

    s1 = _make_sc_scatter(128)(g1, packed, jnp.zeros((RPT, 128), jnp.float32))
    g2 = _tc_mid(s1.reshape(2, N, 128), g1, dinv, b1.reshape(1, -1), W2)

    s2 = _make_sc_scatter(64)(g2, packed, jnp.zeros((RPT, 64), jnp.float32))
    g3 = _tc_mid(s2.reshape(2, N, 64), g2, dinv, b2.reshape(1, -1), W3)

    s3 = _make_sc_scatter(32)(g3, packed, jnp.zeros((RPT, 32), jnp.float32))
    return _tc_final(s3.reshape(2, N, 32), g3, dinv, b3.reshape(1, -1),
                     batch.reshape(N, 1), Wfc, bfc.reshape(1, NUM_CLASSES))


# speedup vs baseline: 1.9423x; 1.6552x over previous
import functools

import jax
import jax.numpy as jnp
from jax import lax
from jax.experimental import pallas as pl
from jax.experimental.pallas import tpu as pltpu, tpu_sc as plsc

N = 10000
E = 320000
NUM_GRAPHS = 64
NUM_CLASSES = 10

NC, NS, LANES = 2, 16, 16
NW = NC * NS
EPT = E // NW
K = 128
NCHUNK = 2560
EP = NCHUNK * K
GROWS = N + 8
_NB = {16: 5, 32: 5, 64: 4, 128: 2}
_CPT0 = {16: 80, 32: 80, 64: 80, 128: 80}
RPT = 1000
NWB = N // RPT

_MESH = plsc.VectorSubcoreMesh(core_axis_name="c", subcore_axis_name="s")


def _make_sc_scatter(D):
    NB = _NB[D]
    CPT0 = _CPT0[D]
    CPT1 = 160 - CPT0
    assert CPT0 % NB == 0 and CPT1 % NB == 0

    @functools.partial(
        pl.kernel,
        out_type=jax.ShapeDtypeStruct((NC * N, D), jnp.float32),
        mesh=_MESH,
        scratch_types=[
            [pltpu.VMEM((2, K), jnp.int32) for _ in range(NB)],
            [pltpu.VMEM((K, D), jnp.float32) for _ in range(NB)],
            pltpu.VMEM_SHARED((N, D), jnp.float32),
            [pltpu.SemaphoreType.DMA for _ in range(NB)],
            [pltpu.SemaphoreType.DMA for _ in range(NB)],
        ],
        compiler_params=pltpu.CompilerParams(use_tc_tiling_on_sc=False),
    )
    def sc_scatter(g_hbm, packed_hbm, zeros_hbm, out_hbm,
                   ibuf, rows, acc, isem, gsem):
        c = lax.axis_index("c")
        s = lax.axis_index("s")
        r0 = s * RPT
        cpt = CPT0 - (CPT0 - CPT1) * c
        cbase = c * NS * CPT0 + s * cpt

        @pl.when(s < NWB)
        def _zero():
            pltpu.sync_copy(zeros_hbm, acc.at[pl.ds(r0, RPT)])

        def load_idx(chunk, b):
            pltpu.async_copy(packed_hbm.at[cbase + chunk], ibuf[b], isem[b])

        def wait_idx(b):
            pltpu.make_async_copy(packed_hbm.at[cbase], ibuf[b],
                                  isem[b]).wait()

        def gather(b):
            pltpu.async_copy(g_hbm.at[ibuf[b].at[0]], rows[b], gsem[b])

        def wait_gather(b):
            pltpu.make_async_copy(g_hbm.at[ibuf[b].at[0]], rows[b],
                                  gsem[b]).wait()

        for b in range(NB):
            load_idx(b, b)
        for b in range(NB - 1):
            wait_idx(b)
            gather(b)
        plsc.subcore_barrier()

        def outer(j, carry):
            for b in range(NB):
                i = NB * j + b
                bprev = (b - 1) % NB
                wait_gather(b)

                @pl.when(i + NB - 1 < cpt)
                def _issue_gather():
                    wait_idx(bprev)
                    gather(bprev)

                pltpu.sync_copy(rows[b], acc.at[ibuf[b].at[1]], add=True)

                @pl.when(i + NB < cpt)
                def _prefetch_idx():
                    load_idx(i + NB, b)
            return carry

        lax.fori_loop(0, cpt // NB, outer, 0)
        plsc.subcore_barrier()

        @pl.when(s < NWB)
        def _writeback():
            pltpu.sync_copy(acc.at[pl.ds(r0, RPT)],
                            out_hbm.at[pl.ds(c * N + r0, RPT)])

    return sc_scatter


RB = 1000


def _tc_first(dp, x, W1):
    D = W1.shape[1]

    def body(dp_ref, x_ref, w_ref, g_ref, dinv_ref):
        d = dp_ref[...]
        deg = d[0, :, 0] + d[1, :, 0] + 1.0
        dinv = lax.rsqrt(deg)
        h = jnp.dot(x_ref[...], w_ref[...], preferred_element_type=jnp.float32)
        g_ref[...] = h * dinv[:, None]
        dinv_ref[...] = dinv[:, None]

    return pl.pallas_call(
        body,
        grid=(N // RB,),
        in_specs=[
            pl.BlockSpec((2, RB, LANES), lambda i: (0, i, 0)),
            pl.BlockSpec((RB, x.shape[1]), lambda i: (i, 0)),
            pl.BlockSpec(W1.shape, lambda i: (0, 0)),
        ],
        out_specs=[
            pl.BlockSpec((RB, D), lambda i: (i, 0)),
            pl.BlockSpec((RB, 1), lambda i: (i, 0)),
        ],
        out_shape=[
            jax.ShapeDtypeStruct((N, D), jnp.float32),
            jax.ShapeDtypeStruct((N, 1), jnp.float32),
        ],
    )(dp, x, W1)


def _tc_mid(s, g, dinv, b, W):
    D = g.shape[1]
    Do = W.shape[1]

    def body(s_ref, g_ref, dinv_ref, b_ref, w_ref, o_ref):
        sp = s_ref[...]
        dv = dinv_ref[...]
        xn = jnp.maximum(dv * (sp[0] + sp[1] + g_ref[...]) + b_ref[...], 0.0)
        h = jnp.dot(xn, w_ref[...], preferred_element_type=jnp.float32)
        o_ref[...] = h * dv

    return pl.pallas_call(
        body,
        grid=(N // RB,),
        in_specs=[
            pl.BlockSpec((2, RB, D), lambda i: (0, i, 0)),
            pl.BlockSpec((RB, D), lambda i: (i, 0)),
            pl.BlockSpec((RB, 1), lambda i: (i, 0)),
            pl.BlockSpec((1, D), lambda i: (0, 0)),
            pl.BlockSpec(W.shape, lambda i: (0, 0)),
        ],
        out_specs=pl.BlockSpec((RB, Do), lambda i: (i, 0)),
        out_shape=jax.ShapeDtypeStruct((N, Do), jnp.float32),
    )(s, g, dinv, b, W)


def _tc_final(s, g, dinv, b, batch, Wfc, bfc):
    D = g.shape[1]

    def body(s_ref, g_ref, dinv_ref, b_ref, bt_ref, wfc_ref, bfc_ref,
             o_ref, pooled_ref):
        sp = s_ref[...]
        x4 = jnp.maximum(
            dinv_ref[...] * (sp[0] + sp[1] + g_ref[...]) + b_ref[...], 0.0)
        bt = bt_ref[...]

        def seg(gi, carry):
            m = bt == gi
            v = jnp.max(jnp.where(m, x4, -jnp.inf), axis=0, keepdims=True)
            pooled_ref[pl.ds(gi, 1), :] = v
            return carry

        lax.fori_loop(0, NUM_GRAPHS, seg, 0)
        logits = jnp.dot(pooled_ref[...], wfc_ref[...],
                         preferred_element_type=jnp.float32) + bfc_ref[...]
        mx = jnp.max(logits, axis=1, keepdims=True)
        sh = logits - mx
        o_ref[...] = sh - jnp.log(jnp.sum(jnp.exp(sh), axis=1, keepdims=True))

    return pl.pallas_call(
        body,
        grid=(1,),
        in_specs=[
            pl.BlockSpec((2, N, D), lambda i: (0, 0, 0)),
            pl.BlockSpec((N, D), lambda i: (0, 0)),
            pl.BlockSpec((N, 1), lambda i: (0, 0)),
            pl.BlockSpec((1, D), lambda i: (0, 0)),
            pl.BlockSpec((N, 1), lambda i: (0, 0)),
            pl.BlockSpec(Wfc.shape, lambda i: (0, 0)),
            pl.BlockSpec((1, NUM_CLASSES), lambda i: (0, 0)),
        ],
        out_specs=pl.BlockSpec((NUM_GRAPHS, NUM_CLASSES), lambda i: (0, 0)),
        out_shape=jax.ShapeDtypeStruct((NUM_GRAPHS, NUM_CLASSES), jnp.float32),
        scratch_shapes=[pltpu.VMEM((NUM_GRAPHS, D), jnp.float32)],
    )(s, g, dinv, b, batch, Wfc, bfc)


def kernel(x, edge_index, batch, W1, b1, W2, b2, W3, b3, Wfc, bfc):
    npad = EP - E
    ar = jnp.arange(npad, dtype=jnp.int32)
    pad = jnp.stack([N + (ar % 8), ar % N])
    packed = (jnp.concatenate([edge_index, pad], axis=1)
              .reshape(2, NCHUNK, K).transpose(1, 0, 2))
    zrows = jnp.zeros((8, 128), jnp.float32)

    def gtable(g):
        return jnp.concatenate([g, zrows[:, :g.shape[1]]], axis=0)

    dp = _make_sc_scatter(LANES)(
        gtable(jnp.ones((N, LANES), jnp.float32)), packed,
        jnp.zeros((RPT, LANES), jnp.float32)).reshape(2, N, LANES)
    g1, dinv = _tc_first(dp, x, W1)

    s1 = _make_sc_scatter(128)(gtable(g1), packed,
                               jnp.zeros((RPT, 128), jnp.float32))
    g2 = _tc_mid(s1.reshape(2, N, 128), g1, dinv, b1.reshape(1, -1), W2)

    s2 = _make_sc_scatter(64)(gtable(g2), packed,
                              jnp.zeros((RPT, 64), jnp.float32))
    g3 = _tc_mid(s2.reshape(2, N, 64), g2, dinv, b2.reshape(1, -1), W3)

    s3 = _make_sc_scatter(32)(gtable(g3), packed,
                              jnp.zeros((RPT, 32), jnp.float32))
    return _tc_final(s3.reshape(2, N, 32), g3, dinv, b3.reshape(1, -1),
                     batch.reshape(N, 1), Wfc, bfc.reshape(1, NUM_CLASSES))


# speedup vs baseline: 2.2520x; 1.1595x over previous
import functools

import jax
import jax.numpy as jnp
from jax import lax
from jax.experimental import pallas as pl
from jax.experimental.pallas import tpu as pltpu, tpu_sc as plsc

N = 10000
E = 320000
NUM_GRAPHS = 64
NUM_CLASSES = 10

NC, NS, LANES = 2, 16, 16
NW = NC * NS
EPT = E // NW
K = 128
NCHUNK = 2560
EP = NCHUNK * K
GROWS = N + 8
_NB = {16: 5, 32: 5, 64: 4, 128: 2}
_CPT0 = {16: 80, 32: 80, 64: 80, 128: 80}
RPT = 1000
NWB = N // RPT

_MESH = plsc.VectorSubcoreMesh(core_axis_name="c", subcore_axis_name="s")


def _make_sc_scatter(D):
    NB = _NB[D]
    CPT0 = _CPT0[D]
    CPT1 = 160 - CPT0
    assert CPT0 % NB == 0 and CPT1 % NB == 0

    @functools.partial(
        pl.kernel,
        out_type=jax.ShapeDtypeStruct((NC * N, D), jnp.float32),
        mesh=_MESH,
        scratch_types=[
            [pltpu.VMEM((2, K), jnp.int32) for _ in range(NB)],
            [pltpu.VMEM((K, D), jnp.float32) for _ in range(NB)],
            pltpu.VMEM_SHARED((N, D), jnp.float32),
            [pltpu.SemaphoreType.DMA for _ in range(NB)],
            [pltpu.SemaphoreType.DMA for _ in range(NB)],
        ],
        compiler_params=pltpu.CompilerParams(use_tc_tiling_on_sc=False),
    )
    def sc_scatter(g_hbm, packed_hbm, zeros_hbm, out_hbm,
                   ibuf, rows, acc, isem, gsem):
        c = lax.axis_index("c")
        s = lax.axis_index("s")
        r0 = s * RPT
        cpt = CPT0 - (CPT0 - CPT1) * c
        cbase = c * NS * CPT0 + s * cpt

        @pl.when(s < NWB)
        def _zero():
            pltpu.sync_copy(zeros_hbm, acc.at[pl.ds(r0, RPT)])

        def load_idx(chunk, b):
            pltpu.async_copy(packed_hbm.at[cbase + chunk], ibuf[b], isem[b])

        def wait_idx(b):
            pltpu.make_async_copy(packed_hbm.at[cbase], ibuf[b],
                                  isem[b]).wait()

        def gather(b):
            pltpu.async_copy(g_hbm.at[ibuf[b].at[0]], rows[b], gsem[b])

        def wait_gather(b):
            pltpu.make_async_copy(g_hbm.at[ibuf[b].at[0]], rows[b],
                                  gsem[b]).wait()

        for b in range(NB):
            load_idx(b, b)
        for b in range(NB - 1):
            wait_idx(b)
            gather(b)
        plsc.subcore_barrier()

        def outer(j, carry):
            for b in range(NB):
                i = NB * j + b
                bprev = (b - 1) % NB
                wait_gather(b)

                @pl.when(i + NB - 1 < cpt)
                def _issue_gather():
                    wait_idx(bprev)
                    gather(bprev)

                pltpu.sync_copy(rows[b], acc.at[ibuf[b].at[1]], add=True)

                @pl.when(i + NB < cpt)
                def _prefetch_idx():
                    load_idx(i + NB, b)
            return carry

        lax.fori_loop(0, cpt // NB, outer, 0)
        plsc.subcore_barrier()

        @pl.when(s < NWB)
        def _writeback():
            pltpu.sync_copy(acc.at[pl.ds(r0, RPT)],
                            out_hbm.at[pl.ds(c * N + r0, RPT)])

    return sc_scatter


RB = 1000


def _tc_first(dp, x, W1):
    D = W1.shape[1]

    def body(d0_ref, d1_ref, x_ref, w_ref, g_ref, dinv_ref):
        deg = d0_ref[...][:, 0] + d1_ref[...][:, 0] + 1.0
        dinv = lax.rsqrt(deg)
        h = jnp.dot(x_ref[...], w_ref[...], preferred_element_type=jnp.float32)
        g_ref[...] = h * dinv[:, None]
        dinv_ref[...] = dinv[:, None]

    nb = N // RB
    return pl.pallas_call(
        body,
        grid=(nb,),
        in_specs=[
            pl.BlockSpec((RB, LANES), lambda i: (i, 0)),
            pl.BlockSpec((RB, LANES), lambda i: (i + nb, 0)),
            pl.BlockSpec((RB, x.shape[1]), lambda i: (i, 0)),
            pl.BlockSpec(W1.shape, lambda i: (0, 0)),
        ],
        out_specs=[
            pl.BlockSpec((RB, D), lambda i: (i, 0)),
            pl.BlockSpec((RB, 1), lambda i: (i, 0)),
        ],
        out_shape=[
            jax.ShapeDtypeStruct((N, D), jnp.float32),
            jax.ShapeDtypeStruct((N, 1), jnp.float32),
        ],
    )(dp, dp, x, W1)


def _tc_mid(s, g, dinv, b, W):
    D = g.shape[1]
    Do = W.shape[1]

    def body(s0_ref, s1_ref, g_ref, dinv_ref, b_ref, w_ref, o_ref):
        dv = dinv_ref[...]
        xn = jnp.maximum(
            dv * (s0_ref[...] + s1_ref[...] + g_ref[...]) + b_ref[...], 0.0)
        h = jnp.dot(xn, w_ref[...], preferred_element_type=jnp.float32)
        o_ref[...] = h * dv

    nb = N // RB
    return pl.pallas_call(
        body,
        grid=(nb,),
        in_specs=[
            pl.BlockSpec((RB, D), lambda i: (i, 0)),
            pl.BlockSpec((RB, D), lambda i: (i + nb, 0)),
            pl.BlockSpec((RB, D), lambda i: (i, 0)),
            pl.BlockSpec((RB, 1), lambda i: (i, 0)),
            pl.BlockSpec((1, D), lambda i: (0, 0)),
            pl.BlockSpec(W.shape, lambda i: (0, 0)),
        ],
        out_specs=pl.BlockSpec((RB, Do), lambda i: (i, 0)),
        out_shape=jax.ShapeDtypeStruct((N, Do), jnp.float32),
    )(s, s, g, dinv, b, W)


def _tc_final(s, g, dinv, b, batch, Wfc, bfc):
    D = g.shape[1]
    NBLK = N // 8
    NINF = float("-inf")

    def body(s0_ref, s1_ref, g_ref, dinv_ref, bt_ref, b_ref, wfc_ref,
             bfc_ref, o_ref, x4_ref):
        x4 = jnp.maximum(
            dinv_ref[...] * (s0_ref[...] + s1_ref[...] + g_ref[...])
            + b_ref[...], 0.0)
        x4_ref[pl.ds(0, N), :] = x4
        x4_ref[pl.ds(N, 8), :] = jnp.full((8, D), NINF, jnp.float32)

        bt = bt_ref[...]
        gid = lax.broadcasted_iota(jnp.int32, (1, NUM_GRAPHS), 1)
        lo = jnp.sum((bt < gid).astype(jnp.int32), axis=0)
        hi = jnp.sum((bt <= gid).astype(jnp.int32), axis=0)

        bm = jnp.max(x4.reshape(NBLK, 8, D), axis=1)
        bidx = lax.broadcasted_iota(jnp.int32, (NBLK, 1), 0)
        rid8 = lax.broadcasted_iota(jnp.int32, (8, 1), 0)

        cols = []
        for gi in range(NUM_GRAPHS):
            lof = lo[gi]
            hif = hi[gi]
            bs = (lof + 7) // 8
            be = hif // 8
            mfull = (bidx >= bs) & (bidx < be)
            vfull = jnp.max(jnp.where(mfull, bm, NINF), axis=0)
            h0 = (lof // 8) * 8
            hr = h0 + rid8
            hm = (hr >= lof) & (hr < hif)
            vh = jnp.max(jnp.where(hm, x4_ref[pl.ds(h0, 8), :], NINF), axis=0)
            t0 = be * 8
            tr = t0 + rid8
            tm = (tr >= lof) & (tr < hif)
            vt = jnp.max(jnp.where(tm, x4_ref[pl.ds(t0, 8), :], NINF), axis=0)
            cols.append(jnp.maximum(jnp.maximum(vfull, vh), vt)[None, :])
        pooled = jnp.concatenate(cols, axis=0)

        logits = jnp.dot(pooled, wfc_ref[...],
                         preferred_element_type=jnp.float32) + bfc_ref[...]
        mx = jnp.max(logits, axis=1, keepdims=True)
        sh = logits - mx
        o_ref[...] = sh - jnp.log(jnp.sum(jnp.exp(sh), axis=1, keepdims=True))

    return pl.pallas_call(
        body,
        grid=(1,),
        in_specs=[
            pl.BlockSpec((N, D), lambda i: (0, 0)),
            pl.BlockSpec((N, D), lambda i: (1, 0)),
            pl.BlockSpec((N, D), lambda i: (0, 0)),
            pl.BlockSpec((N, 1), lambda i: (0, 0)),
            pl.BlockSpec((N, 1), lambda i: (0, 0)),
            pl.BlockSpec((1, D), lambda i: (0, 0)),
            pl.BlockSpec(Wfc.shape, lambda i: (0, 0)),
            pl.BlockSpec((1, NUM_CLASSES), lambda i: (0, 0)),
        ],
        out_specs=pl.BlockSpec((NUM_GRAPHS, NUM_CLASSES), lambda i: (0, 0)),
        out_shape=jax.ShapeDtypeStruct((NUM_GRAPHS, NUM_CLASSES), jnp.float32),
        scratch_shapes=[pltpu.VMEM((N + 8, D), jnp.float32)],
    )(s, s, g, dinv, batch, b, Wfc, bfc)


def kernel(x, edge_index, batch, W1, b1, W2, b2, W3, b3, Wfc, bfc):
    npad = EP - E
    ar = jnp.arange(npad, dtype=jnp.int32)
    pad = jnp.stack([N + (ar % 8), ar % N])
    packed = (jnp.concatenate([edge_index, pad], axis=1)
              .reshape(2, NCHUNK, K).transpose(1, 0, 2))
    zrows = jnp.zeros((8, 128), jnp.float32)

    def gtable(g):
        return jnp.concatenate([g, zrows[:, :g.shape[1]]], axis=0)

    dp = _make_sc_scatter(LANES)(
        gtable(jnp.ones((N, LANES), jnp.float32)), packed,
        jnp.zeros((RPT, LANES), jnp.float32))
    g1, dinv = _tc_first(dp, x, W1)

    s1 = _make_sc_scatter(128)(gtable(g1), packed,
                               jnp.zeros((RPT, 128), jnp.float32))
    g2 = _tc_mid(s1, g1, dinv, b1.reshape(1, -1), W2)

    s2 = _make_sc_scatter(64)(gtable(g2), packed,
                              jnp.zeros((RPT, 64), jnp.float32))
    g3 = _tc_mid(s2, g2, dinv, b2.reshape(1, -1), W3)

    s3 = _make_sc_scatter(32)(gtable(g3), packed,
                              jnp.zeros((RPT, 32), jnp.float32))
    return _tc_final(s3, g3, dinv, b3.reshape(1, -1),
                     batch.reshape(N, 1), Wfc, bfc.reshape(1, NUM_CLASSES))


# speedup vs baseline: 2.5928x; 1.1513x over previous
import functools

import jax
import jax.numpy as jnp
from jax import lax
from jax.experimental import pallas as pl
from jax.experimental.pallas import tpu as pltpu, tpu_sc as plsc

N = 10000
E = 320000
NUM_GRAPHS = 64
NUM_CLASSES = 10

NC, NS, LANES = 2, 16, 16
NW = NC * NS
EPT = E // NW
K = 128
NCHUNK = 2560
EP = NCHUNK * K
GROWS = N + 8
SPARE = 128
_NB = {16: 5, 32: 5, 64: 4, 128: 2}
_CPT0 = {16: 85, 32: 85, 64: 76, 128: 84}
RPT = 1000
NWB = N // RPT

_MESH = plsc.VectorSubcoreMesh(core_axis_name="c", subcore_axis_name="s")


def _make_sc_scatter(D, ones_src=False):
    NB = _NB[D]
    CPT0 = _CPT0[D]
    CPT1 = 160 - CPT0
    assert CPT0 % NB == 0 and CPT1 % NB == 0

    @functools.partial(
        pl.kernel,
        out_type=jax.ShapeDtypeStruct((NC * N, D), jnp.float32),
        mesh=_MESH,
        scratch_types=[
            [pltpu.VMEM((2, K), jnp.int32) for _ in range(NB)],
            [pltpu.VMEM((K, D), jnp.float32) for _ in range(NB)],
            pltpu.VMEM_SHARED((N + SPARE, D), jnp.float32),
            [pltpu.SemaphoreType.DMA for _ in range(NB)],
            [pltpu.SemaphoreType.DMA for _ in range(NB)],
        ],
        compiler_params=pltpu.CompilerParams(use_tc_tiling_on_sc=False),
    )
    def sc_scatter(g_hbm, packed_hbm, zeros_hbm, out_hbm,
                   ibuf, rows, acc, isem, gsem):
        c = lax.axis_index("c")
        s = lax.axis_index("s")
        r0 = s * RPT
        cpt = CPT0 - (CPT0 - CPT1) * c
        cbase = c * NS * CPT0 + s * cpt

        @pl.when(s < NWB)
        def _zero():
            pltpu.sync_copy(zeros_hbm, acc.at[pl.ds(r0, RPT)])

        def load_idx(chunk, b):
            pltpu.async_copy(packed_hbm.at[cbase + chunk], ibuf[b], isem[b])

        def wait_idx(b):
            pltpu.make_async_copy(packed_hbm.at[cbase], ibuf[b],
                                  isem[b]).wait()

        def gather(b):
            pltpu.async_copy(g_hbm.at[ibuf[b].at[0]], rows[b], gsem[b])

        def wait_gather(b):
            pltpu.make_async_copy(g_hbm.at[ibuf[b].at[0]], rows[b],
                                  gsem[b]).wait()

        for b in range(NB):
            load_idx(b, b)

        if ones_src:
            one = jnp.ones((LANES,), jnp.float32)

            def fill(r, carry):
                for b in range(NB):
                    for col in range(D // LANES):
                        rows[b][r, pl.ds(col * LANES, LANES)] = one
                return carry

            lax.fori_loop(0, K, fill, 0)
            plsc.subcore_barrier()

            def outer_ones(j, carry):
                for b in range(NB):
                    i = NB * j + b
                    wait_idx(b)
                    pltpu.sync_copy(rows[b], acc.at[ibuf[b].at[1]], add=True)

                    @pl.when(i + NB < cpt)
                    def _prefetch_idx():
                        load_idx(i + NB, b)
                return carry

            lax.fori_loop(0, cpt // NB, outer_ones, 0)
        else:
            for b in range(NB - 1):
                wait_idx(b)
                gather(b)
            plsc.subcore_barrier()

            def outer(j, carry):
                for b in range(NB):
                    i = NB * j + b
                    bprev = (b - 1) % NB
                    wait_gather(b)

                    @pl.when(i + NB - 1 < cpt)
                    def _issue_gather():
                        wait_idx(bprev)
                        gather(bprev)

                    pltpu.sync_copy(rows[b], acc.at[ibuf[b].at[1]], add=True)

                    @pl.when(i + NB < cpt)
                    def _prefetch_idx():
                        load_idx(i + NB, b)
                return carry

            lax.fori_loop(0, cpt // NB, outer, 0)

        plsc.subcore_barrier()

        @pl.when(s < NWB)
        def _writeback():
            pltpu.sync_copy(acc.at[pl.ds(r0, RPT)],
                            out_hbm.at[pl.ds(c * N + r0, RPT)])

    return sc_scatter


RB = 1000


def _tc_first(dp, x, W1):
    D = W1.shape[1]

    def body(d0_ref, d1_ref, x_ref, w_ref, g_ref, dinv_ref):
        deg = d0_ref[...][:, 0] + d1_ref[...][:, 0] + 1.0
        dinv = lax.rsqrt(deg)
        h = jnp.dot(x_ref[...], w_ref[...], preferred_element_type=jnp.float32)
        g_ref[...] = h * dinv[:, None]
        dinv_ref[...] = dinv[:, None]

    nb = N // RB
    return pl.pallas_call(
        body,
        grid=(nb,),
        in_specs=[
            pl.BlockSpec((RB, LANES), lambda i: (i, 0)),
            pl.BlockSpec((RB, LANES), lambda i: (i + nb, 0)),
            pl.BlockSpec((RB, x.shape[1]), lambda i: (i, 0)),
            pl.BlockSpec(W1.shape, lambda i: (0, 0)),
        ],
        out_specs=[
            pl.BlockSpec((RB, D), lambda i: (i, 0)),
            pl.BlockSpec((RB, 1), lambda i: (i, 0)),
        ],
        out_shape=[
            jax.ShapeDtypeStruct((N, D), jnp.float32),
            jax.ShapeDtypeStruct((N, 1), jnp.float32),
        ],
    )(dp, dp, x, W1)


def _tc_mid(s, g, dinv, b, W):
    D = g.shape[1]
    Do = W.shape[1]

    def body(s0_ref, s1_ref, g_ref, dinv_ref, b_ref, w_ref, o_ref):
        dv = dinv_ref[...]
        xn = jnp.maximum(
            dv * (s0_ref[...] + s1_ref[...] + g_ref[...]) + b_ref[...], 0.0)
        h = jnp.dot(xn, w_ref[...], preferred_element_type=jnp.float32)
        o_ref[...] = h * dv

    nb = N // RB
    return pl.pallas_call(
        body,
        grid=(nb,),
        in_specs=[
            pl.BlockSpec((RB, D), lambda i: (i, 0)),
            pl.BlockSpec((RB, D), lambda i: (i + nb, 0)),
            pl.BlockSpec((RB, D), lambda i: (i, 0)),
            pl.BlockSpec((RB, 1), lambda i: (i, 0)),
            pl.BlockSpec((1, D), lambda i: (0, 0)),
            pl.BlockSpec(W.shape, lambda i: (0, 0)),
        ],
        out_specs=pl.BlockSpec((RB, Do), lambda i: (i, 0)),
        out_shape=jax.ShapeDtypeStruct((N, Do), jnp.float32),
    )(s, s, g, dinv, b, W)


def _tc_final(s, g, dinv, b, batch, Wfc, bfc):
    D = g.shape[1]
    NBLK = N // 8
    NINF = float("-inf")

    def body(s0_ref, s1_ref, g_ref, dinv_ref, bt_ref, b_ref, wfc_ref,
             bfc_ref, o_ref, x4_ref):
        x4 = jnp.maximum(
            dinv_ref[...] * (s0_ref[...] + s1_ref[...] + g_ref[...])
            + b_ref[...], 0.0)
        x4_ref[pl.ds(0, N), :] = x4
        x4_ref[pl.ds(N, 8), :] = jnp.full((8, D), NINF, jnp.float32)

        bt = bt_ref[...]
        gid = lax.broadcasted_iota(jnp.int32, (1, NUM_GRAPHS), 1)
        lo = jnp.sum((bt < gid).astype(jnp.int32), axis=0)
        hi = jnp.sum((bt <= gid).astype(jnp.int32), axis=0)

        bm = jnp.max(x4.reshape(NBLK, 8, D), axis=1)
        bidx = lax.broadcasted_iota(jnp.int32, (NBLK, 1), 0)
        rid8 = lax.broadcasted_iota(jnp.int32, (8, 1), 0)

        cols = []
        for gi in range(NUM_GRAPHS):
            lof = lo[gi]
            hif = hi[gi]
            bs = (lof + 7) // 8
            be = hif // 8
            mfull = (bidx >= bs) & (bidx < be)
            vfull = jnp.max(jnp.where(mfull, bm, NINF), axis=0)
            h0 = (lof // 8) * 8
            hr = h0 + rid8
            hm = (hr >= lof) & (hr < hif)
            vh = jnp.max(jnp.where(hm, x4_ref[pl.ds(h0, 8), :], NINF), axis=0)
            t0 = be * 8
            tr = t0 + rid8
            tm = (tr >= lof) & (tr < hif)
            vt = jnp.max(jnp.where(tm, x4_ref[pl.ds(t0, 8), :], NINF), axis=0)
            cols.append(jnp.maximum(jnp.maximum(vfull, vh), vt)[None, :])
        pooled = jnp.concatenate(cols, axis=0)

        logits = jnp.dot(pooled, wfc_ref[...],
                         preferred_element_type=jnp.float32) + bfc_ref[...]
        mx = jnp.max(logits, axis=1, keepdims=True)
        sh = logits - mx
        o_ref[...] = sh - jnp.log(jnp.sum(jnp.exp(sh), axis=1, keepdims=True))

    return pl.pallas_call(
        body,
        grid=(1,),
        in_specs=[
            pl.BlockSpec((N, D), lambda i: (0, 0)),
            pl.BlockSpec((N, D), lambda i: (1, 0)),
            pl.BlockSpec((N, D), lambda i: (0, 0)),
            pl.BlockSpec((N, 1), lambda i: (0, 0)),
            pl.BlockSpec((N, 1), lambda i: (0, 0)),
            pl.BlockSpec((1, D), lambda i: (0, 0)),
            pl.BlockSpec(Wfc.shape, lambda i: (0, 0)),
            pl.BlockSpec((1, NUM_CLASSES), lambda i: (0, 0)),
        ],
        out_specs=pl.BlockSpec((NUM_GRAPHS, NUM_CLASSES), lambda i: (0, 0)),
        out_shape=jax.ShapeDtypeStruct((NUM_GRAPHS, NUM_CLASSES), jnp.float32),
        scratch_shapes=[pltpu.VMEM((N + 8, D), jnp.float32)],
    )(s, s, g, dinv, batch, b, Wfc, bfc)


def kernel(x, edge_index, batch, W1, b1, W2, b2, W3, b3, Wfc, bfc):
    npad = EP - E
    ar = jnp.arange(npad, dtype=jnp.int32)
    pad = jnp.stack([N + (ar % 8), N + (ar % SPARE)])
    packed = (jnp.concatenate([edge_index, pad], axis=1)
              .reshape(2, NCHUNK, K).transpose(1, 0, 2))
    zrows = jnp.zeros((8, 128), jnp.float32)

    def gtable(g):
        return jnp.concatenate([g, zrows[:, :g.shape[1]]], axis=0)

    dp = _make_sc_scatter(LANES, ones_src=True)(
        jnp.zeros((8, LANES), jnp.float32), packed,
        jnp.zeros((RPT, LANES), jnp.float32))
    g1, dinv = _tc_first(dp, x, W1)

    s1 = _make_sc_scatter(128)(gtable(g1), packed,
                               jnp.zeros((RPT, 128), jnp.float32))
    g2 = _tc_mid(s1, g1, dinv, b1.reshape(1, -1), W2)

    s2 = _make_sc_scatter(64)(gtable(g2), packed,
                              jnp.zeros((RPT, 64), jnp.float32))
    g3 = _tc_mid(s2, g2, dinv, b2.reshape(1, -1), W3)

    s3 = _make_sc_scatter(32)(gtable(g3), packed,
                              jnp.zeros((RPT, 32), jnp.float32))
    return _tc_final(s3, g3, dinv, b3.reshape(1, -1),
                     batch.reshape(N, 1), Wfc, bfc.reshape(1, NUM_CLASSES))
